# speedup vs baseline: 1.4308x; 1.4308x over previous
import functools

import jax
import jax.numpy as jnp
from jax import lax
from jax.experimental import pallas as pl
from jax.experimental.pallas import tpu as pltpu
from jax.experimental.pallas import tpu_sc as plsc

EPS = 1e-5
NC = 2
NS = 16
NW = NC * NS
CHUNK = 64


def _sc_gather(table, idx3, hidden):
    nw, n_chunks, chunk = idx3.shape
    n = nw * n_chunks * chunk
    rows_per_tile = n_chunks * chunk
    mesh = plsc.VectorSubcoreMesh(core_axis_name="c", subcore_axis_name="s")

    @functools.partial(
        pl.kernel,
        mesh=mesh,
        out_type=jax.ShapeDtypeStruct((n, hidden), jnp.float32),
        scratch_types=[
            pltpu.VMEM((n_chunks, chunk), jnp.int32),
            pltpu.VMEM((chunk, hidden), jnp.float32),
            pltpu.SemaphoreType.DMA,
        ],
    )
    def k(table_hbm, idx_hbm, out_hbm, idx_v, rows_v, sem):
        wid = lax.axis_index("s") * NC + lax.axis_index("c")
        base = wid * rows_per_tile
        pltpu.sync_copy(idx_hbm.at[wid], idx_v)

        @pl.loop(0, n_chunks)
        def _(c):
            pltpu.async_copy(table_hbm.at[idx_v.at[c]], rows_v, sem).wait()
            pltpu.sync_copy(rows_v, out_hbm.at[pl.ds(base + c * chunk, chunk)])

    return k(table, idx3)


def _ln_body(g_ref, p_ref, w_ref, b_ref, o_ref):
    x = g_ref[...] + p_ref[...]
    m = jnp.mean(x, axis=-1, keepdims=True)
    xc = x - m
    v = jnp.mean(xc * xc, axis=-1, keepdims=True)
    o_ref[...] = xc * lax.rsqrt(v + EPS) * w_ref[...] + b_ref[...]


def kernel(input_ids, embed_tokens, embed_positions, ln_weight, ln_bias):
    batch, seq = input_ids.shape
    vocab, hidden = embed_tokens.shape
    n = batch * seq

    idx = input_ids.reshape(-1).astype(jnp.int32)
    idx3 = idx.reshape(NW, -1, CHUNK)
    gathered = _sc_gather(embed_tokens, idx3, hidden)

    blk = 512
    pos_blocks = seq // blk
    out = pl.pallas_call(
        _ln_body,
        grid=(n // blk,),
        in_specs=[
            pl.BlockSpec((blk, hidden), lambda i: (i, 0)),
            pl.BlockSpec((blk, hidden), lambda i: (i % pos_blocks, 0)),
            pl.BlockSpec((1, hidden), lambda i: (0, 0)),
            pl.BlockSpec((1, hidden), lambda i: (0, 0)),
        ],
        out_specs=pl.BlockSpec((blk, hidden), lambda i: (i, 0)),
        out_shape=jax.ShapeDtypeStruct((n, hidden), jnp.float32),
    )(gathered, embed_positions, ln_weight.reshape(1, hidden),
      ln_bias.reshape(1, hidden))
    return out.reshape(batch, seq, hidden)


# speedup vs baseline: 1.5137x; 1.0579x over previous
import functools

import jax
import jax.numpy as jnp
from jax import lax
from jax.experimental import pallas as pl
from jax.experimental.pallas import tpu as pltpu
from jax.experimental.pallas import tpu_sc as plsc

EPS = 1e-5
NC = 2
NS = 16
NW = NC * NS
SUB = 32
N_CHUNKS = 4
BLK = 512


def _sc_gather(table, idx3, hidden):
    nw, nsub, sub = idx3.shape
    n = nw * nsub * sub
    rows_per_tile = nsub * sub
    mesh = plsc.VectorSubcoreMesh(core_axis_name="c", subcore_axis_name="s")

    @functools.partial(
        pl.kernel,
        mesh=mesh,
        out_type=jax.ShapeDtypeStruct((n, hidden), jnp.float32),
        scratch_types=[
            pltpu.VMEM((nsub, sub), jnp.int32),
            pltpu.VMEM((sub, hidden), jnp.float32),
            pltpu.VMEM((sub, hidden), jnp.float32),
            pltpu.SemaphoreType.DMA,
            pltpu.SemaphoreType.DMA,
            pltpu.SemaphoreType.DMA,
            pltpu.SemaphoreType.DMA,
        ],
    )
    def k(table_hbm, idx_hbm, out_hbm, idx_v, buf0, buf1, g0, g1, s0, s1):
        wid = lax.axis_index("s") * NC + lax.axis_index("c")
        base = wid * rows_per_tile
        pltpu.sync_copy(idx_hbm.at[wid], idx_v)
        bufs = (buf0, buf1)
        gsems = (g0, g1)
        ssems = (s0, s1)

        pend = [None] * nsub
        pend[0] = pltpu.async_copy(table_hbm.at[idx_v.at[0]], buf0, g0)
        if nsub > 1:
            pend[1] = pltpu.async_copy(table_hbm.at[idx_v.at[1]], buf1, g1)

        for c in range(nsub):
            b = c % 2
            pend[c].wait()
            st = pltpu.async_copy(
                bufs[b], out_hbm.at[pl.ds(base + c * sub, sub)], ssems[b]
            )
            st.wait()
            if c + 2 < nsub:
                pend[c + 2] = pltpu.async_copy(
                    table_hbm.at[idx_v.at[c + 2]], bufs[b], gsems[b]
                )

    return k(table, idx3)


def _ln_body(prev_ref, g_ref, p_ref, w_ref, b_ref, o_ref):
    del prev_ref
    x = g_ref[...] + p_ref[...]
    m = jnp.mean(x, axis=-1, keepdims=True)
    xc = x - m
    v = jnp.mean(xc * xc, axis=-1, keepdims=True)
    o_ref[...] = xc * lax.rsqrt(v + EPS) * w_ref[...] + b_ref[...]


def kernel(input_ids, embed_tokens, embed_positions, ln_weight, ln_bias):
    batch, seq = input_ids.shape
    vocab, hidden = embed_tokens.shape
    n = batch * seq
    seq_c = seq // N_CHUNKS
    chunk_rows = batch * seq_c
    pos_blocks = seq_c // BLK

    ids32 = input_ids.astype(jnp.int32)
    w2 = ln_weight.reshape(1, hidden)
    b2 = ln_bias.reshape(1, hidden)

    gathered = []
    for k in range(N_CHUNKS):
        ids_ck = ids32[:, k * seq_c:(k + 1) * seq_c].reshape(NW, -1, SUB)
        gathered.append(_sc_gather(embed_tokens, ids_ck, hidden))

    out = None
    for k in range(N_CHUNKS):
        def g_map(p, b):
            return (b * (seq_c // BLK) + p, 0)

        def p_map(p, b, _k=k):
            return (_k * pos_blocks + p, 0)

        def o_map(p, b, _k=k):
            return (b * (seq // BLK) + _k * pos_blocks + p, 0)

        in_specs = [
            pl.BlockSpec(memory_space=pl.ANY),
            pl.BlockSpec((BLK, hidden), g_map),
            pl.BlockSpec((BLK, hidden), p_map),
            pl.BlockSpec((1, hidden), lambda p, b: (0, 0)),
            pl.BlockSpec((1, hidden), lambda p, b: (0, 0)),
        ]
        if out is None:
            prev = jnp.zeros((8, 128), dtype=jnp.float32)
            in_specs[0] = pl.BlockSpec(memory_space=pl.ANY)
        else:
            prev = out
        out = pl.pallas_call(
            _ln_body,
            grid=(pos_blocks, batch),
            in_specs=in_specs,
            out_specs=pl.BlockSpec((BLK, hidden), o_map),
            out_shape=jax.ShapeDtypeStruct((n, hidden), jnp.float32),
            input_output_aliases={} if out is None else {0: 0},
        )(prev, gathered[k], embed_positions, w2, b2)
    return out.reshape(batch, seq, hidden)
